# baseline (speedup 1.0000x reference)
import jax
import jax.numpy as jnp
from jax.experimental import pallas as pl
from jax.experimental.pallas import tpu as pltpu

_G = 8
_H = 64
_W = 64
_P = _H * _W
_C = 256
_CO = _G * _G + 1


def _shift_rows(y, off):
    if off > 0:
        return jnp.pad(y[off:, :], ((0, off), (0, 0)))
    if off < 0:
        return jnp.pad(y[:off, :], ((-off, 0), (0, 0)))
    return y


def _body(x_ref, w1_ref, b1_ref, w2_ref, b2_ref, out_ref, p_ref):
    n_img = pl.num_programs(0) - 1
    i = pl.program_id(0)
    hb = _H // _G

    def _shuffle_chunk(t):
        @pl.when(i > 0)
        def _():
            v = p_ref[pl.ds(t * hb * _W, hb * _W), :]
            v = v.reshape(hb, _W, _G, _G)
            z = jnp.transpose(v, (0, 2, 1, 3)).reshape(hb * _G, _W * _G)
            out_ref[0, pl.ds(t * hb * _G, hb * _G), :] = z.astype(jnp.float32)

    @pl.when(i < n_img)
    def _compute():
        x = x_ref[0].astype(jnp.bfloat16)
        col = jax.lax.rem(jax.lax.broadcasted_iota(jnp.int32, (_P, 1), 0), _W)
        xl = jnp.where(col == _W - 1, jnp.bfloat16(0), x)
        xr = jnp.where(col == 0, jnp.bfloat16(0), x)
        srcs = (xl, x, xr)
        acc = jnp.zeros((_P, _C), jnp.float32)
        for ky in range(3):
            for kx in range(3):
                y = jnp.dot(srcs[kx], w1_ref[ky * 3 + kx],
                            preferred_element_type=jnp.float32)
                acc = acc + _shift_rows(y, (ky - 1) * _W + (kx - 1))
                t = ky * 3 + kx
                if t < _G:
                    _shuffle_chunk(t)
        hid = jnp.maximum(acc + b1_ref[...], 0.0).astype(jnp.bfloat16)
        logits = jnp.dot(hid, w2_ref[...],
                         preferred_element_type=jnp.float32) + b2_ref[...]
        m = jnp.max(logits, axis=1, keepdims=True)
        e = jnp.exp(logits - m)
        s = jnp.dot(e.astype(jnp.bfloat16), jnp.ones((_CO, 1), jnp.bfloat16),
                    preferred_element_type=jnp.float32)
        p = e * (1.0 / s)
        p_ref[...] = p[:, :_G * _G].astype(jnp.bfloat16)

    @pl.when(i >= n_img)
    def _tail():
        for t in range(_G):
            _shuffle_chunk(t)


def kernel(features, W1, b1, W2, b2):
    n = features.shape[0]
    xt = features.reshape(n, _C, _P).transpose(0, 2, 1)
    w1t = W1.transpose(2, 3, 1, 0).reshape(9, _C, _C).astype(jnp.bfloat16)
    w2t = W2.reshape(_CO, _C).T.astype(jnp.bfloat16)
    out = pl.pallas_call(
        _body,
        grid=(n + 1,),
        in_specs=[
            pl.BlockSpec((1, _P, _C), lambda i: (jnp.minimum(i, n - 1), 0, 0)),
            pl.BlockSpec((9, _C, _C), lambda i: (0, 0, 0)),
            pl.BlockSpec((1, _C), lambda i: (0, 0)),
            pl.BlockSpec((_C, _CO), lambda i: (0, 0)),
            pl.BlockSpec((1, _CO), lambda i: (0, 0)),
        ],
        out_specs=pl.BlockSpec((1, _H * _G, _W * _G),
                               lambda i: (jnp.maximum(i - 1, 0), 0, 0)),
        out_shape=jax.ShapeDtypeStruct((n, _H * _G, _W * _G), jnp.float32),
        scratch_shapes=[pltpu.VMEM((_P, _G * _G), jnp.bfloat16)],
    )(xt, w1t, b1.reshape(1, _C), w2t, b2.reshape(1, _CO))
    return out.reshape(n, 1, _H * _G, _W * _G)


# speedup vs baseline: 1.3995x; 1.3995x over previous
import jax
import jax.numpy as jnp
from jax.experimental import pallas as pl
from jax.experimental.pallas import tpu as pltpu

_G = 8
_H = 64
_W = 64
_P = _H * _W
_C = 256
_CO = _G * _G + 1


def _shift_rows(y, off):
    if off > 0:
        return jnp.pad(y[off:, :], ((0, off), (0, 0)))
    if off < 0:
        return jnp.pad(y[:off, :], ((-off, 0), (0, 0)))
    return y


def _body(x_ref, w1_ref, b1_ref, w2_ref, b2_ref, out_ref, p_ref):
    n_img = pl.num_programs(0) - 1
    i = pl.program_id(0)

    @pl.when(i > 0)
    def _shuffle():
        v = p_ref[...].reshape(_H, _W, _G, _G)
        t = jnp.transpose(v, (0, 2, 1, 3)).reshape(_H * _G, _W * _G)
        out_ref[0] = t.astype(jnp.float32)

    @pl.when(i < n_img)
    def _compute():
        x = x_ref[0].astype(jnp.bfloat16)
        col = jax.lax.rem(jax.lax.broadcasted_iota(jnp.int32, (_P, 1), 0), _W)
        xl = jnp.where(col == _W - 1, jnp.bfloat16(0), x)
        xr = jnp.where(col == 0, jnp.bfloat16(0), x)
        srcs = (xl, x, xr)
        acc = jnp.zeros((_P, _C), jnp.float32)
        for ky in range(3):
            for kx in range(3):
                y = jnp.dot(srcs[kx], w1_ref[ky * 3 + kx],
                            preferred_element_type=jnp.float32)
                acc = acc + _shift_rows(y, (ky - 1) * _W + (kx - 1))
        hid = jnp.maximum(acc + b1_ref[...], 0.0).astype(jnp.bfloat16)
        logits = jnp.dot(hid, w2_ref[...],
                         preferred_element_type=jnp.float32) + b2_ref[...]
        m = jnp.max(logits, axis=1, keepdims=True)
        e = jnp.exp(logits - m)
        p = e / jnp.sum(e, axis=1, keepdims=True)
        p_ref[...] = p[:, :_G * _G].astype(jnp.bfloat16)


def kernel(features, W1, b1, W2, b2):
    n = features.shape[0]
    xt = features.reshape(n, _C, _P).transpose(0, 2, 1)
    w1t = W1.transpose(2, 3, 1, 0).reshape(9, _C, _C).astype(jnp.bfloat16)
    w2t = W2.reshape(_CO, _C).T.astype(jnp.bfloat16)
    out = pl.pallas_call(
        _body,
        grid=(n + 1,),
        in_specs=[
            pl.BlockSpec((1, _P, _C), lambda i: (jnp.minimum(i, n - 1), 0, 0)),
            pl.BlockSpec((9, _C, _C), lambda i: (0, 0, 0)),
            pl.BlockSpec((1, _C), lambda i: (0, 0)),
            pl.BlockSpec((_C, _CO), lambda i: (0, 0)),
            pl.BlockSpec((1, _CO), lambda i: (0, 0)),
        ],
        out_specs=pl.BlockSpec((1, _H * _G, _W * _G),
                               lambda i: (jnp.maximum(i - 1, 0), 0, 0)),
        out_shape=jax.ShapeDtypeStruct((n, _H * _G, _W * _G), jnp.float32),
        scratch_shapes=[pltpu.VMEM((_P, _G * _G), jnp.bfloat16)],
    )(xt, w1t, b1.reshape(1, _C), w2t, b2.reshape(1, _CO))
    return out.reshape(n, 1, _H * _G, _W * _G)
